# Initial kernel scaffold; baseline (speedup 1.0000x reference)
import jax
import jax.numpy as jnp
from jax.experimental import pallas as pl


def kernel(x, shared_in_w, shared_out_w, w1_shared_w, w2_expert_w, group_gate_w, expert_gate_w, group_bias, expert_bias):
    raise NotImplementedError("write your pallas kernel here")


# speedup vs baseline: 20.6045x; 20.6045x over previous
import jax
import jax.numpy as jnp
from jax.experimental import pallas as pl

MODEL_DIM = 768
HIDDEN = 256
NUM_GROUPS = 2
EPG = 4
NUM_EXPERTS = NUM_GROUPS * EPG
TOP_K = 2

TOKEN_TILE = 512


def _moe_kernel(x_ref, w_in_ref, w_out_ref, gates_ref, bias_ref, out_ref):
    xt = x_ref[...]

    logits_t = jax.lax.dot_general(
        gates_ref[...], xt, (((0,), (1,)), ((), ())),
        preferred_element_type=jnp.float32)
    logits_t = logits_t + bias_ref[...]
    gl = logits_t[:NUM_GROUPS, :]
    el = logits_t[NUM_GROUPS:, :]

    gm = jnp.max(gl, axis=0, keepdims=True)
    ge = jnp.exp(gl - gm)
    gp = ge / jnp.sum(ge, axis=0, keepdims=True)
    p0 = gp[0:1, :]
    p1 = gp[1:2, :]
    g1 = (p1 > p0).astype(jnp.int32)
    g_prob = jnp.where(g1 > 0, p1, p0)

    row = jax.lax.broadcasted_iota(jnp.int32, el.shape, 0)
    in_group = (row // EPG) == g1
    neg = jnp.float32(-1e30)
    el_m = jnp.where(in_group, el, neg)
    em = jnp.max(el_m, axis=0, keepdims=True)
    ee = jnp.exp(el_m - em)
    ep = ee / jnp.sum(ee, axis=0, keepdims=True)
    p_exp = ep * g_prob

    rank = jnp.zeros_like(p_exp)
    for j in range(NUM_EXPERTS):
        pj = p_exp[j:j + 1, :]
        rank = rank + (pj > p_exp).astype(jnp.float32)
        rank = rank + ((pj == p_exp) & (row > j)).astype(jnp.float32)
    c_t = jnp.where(rank < TOP_K, p_exp, 0.0)
    c = c_t.T.astype(jnp.bfloat16)

    xb = xt.astype(jnp.bfloat16)
    big1 = jnp.dot(xb, w_in_ref[...], preferred_element_type=jnp.float32)
    hs = big1[:, :2 * HIDDEN]
    h1 = big1[:, 2 * HIDDEN:]
    a = hs[:, :HIDDEN]
    b = hs[:, HIDDEN:]
    hid = ((a * jax.nn.sigmoid(a)) * b).astype(jnp.bfloat16)
    a1 = h1[:, :HIDDEN]
    b1 = h1[:, HIDDEN:]
    h_all = ((a1 * jax.nn.sigmoid(a1)) * b1).astype(jnp.bfloat16)

    parts = [hid] + [h_all * c[:, e:e + 1] for e in range(NUM_EXPERTS)]
    scaled = jnp.concatenate(parts, axis=1)
    out_ref[...] = jnp.dot(scaled, w_out_ref[...],
                           preferred_element_type=jnp.float32)


def kernel(x, shared_in_w, shared_out_w, w1_shared_w, w2_expert_w,
           group_gate_w, expert_gate_w, group_bias, expert_bias):
    Bb, Tt, C = x.shape
    S = Bb * Tt
    flat = x.reshape(S, C)
    gates = jnp.concatenate([group_gate_w, expert_gate_w], axis=1)
    bias = jnp.concatenate([group_bias, expert_bias]).reshape(
        NUM_GROUPS + NUM_EXPERTS, 1)

    w_in = jnp.concatenate([shared_in_w, w1_shared_w],
                           axis=1).astype(jnp.bfloat16)
    w_out = jnp.concatenate(
        [shared_out_w, w2_expert_w.reshape(NUM_EXPERTS * HIDDEN, C)],
        axis=0).astype(jnp.bfloat16)

    grid = (S // TOKEN_TILE,)
    full = lambda *shape: pl.BlockSpec(shape, lambda i: (0,) * len(shape))

    out = pl.pallas_call(
        _moe_kernel,
        grid=grid,
        in_specs=[
            pl.BlockSpec((TOKEN_TILE, C), lambda i: (i, 0)),
            full(C, 4 * HIDDEN),
            full((1 + NUM_EXPERTS) * HIDDEN, C),
            full(C, NUM_GROUPS + NUM_EXPERTS),
            full(NUM_GROUPS + NUM_EXPERTS, 1),
        ],
        out_specs=pl.BlockSpec((TOKEN_TILE, C), lambda i: (i, 0)),
        out_shape=jax.ShapeDtypeStruct((S, C), jnp.float32),
    )(flat, w_in, w_out, gates, bias)

    return out.reshape(Bb, Tt, C)


# speedup vs baseline: 20.8514x; 1.0120x over previous
import jax
import jax.numpy as jnp
from jax.experimental import pallas as pl

MODEL_DIM = 768
HIDDEN = 256
NUM_GROUPS = 2
EPG = 4
NUM_EXPERTS = NUM_GROUPS * EPG
TOP_K = 2

TOKEN_TILE = 1024


def _moe_kernel(x_ref, w_in_ref, w_out_ref, gates_ref, bias_ref, out_ref):
    xt = x_ref[...]

    logits_t = jax.lax.dot_general(
        gates_ref[...], xt, (((0,), (1,)), ((), ())),
        preferred_element_type=jnp.float32)
    logits_t = logits_t + bias_ref[...]
    gl = logits_t[:NUM_GROUPS, :]
    el = logits_t[NUM_GROUPS:, :]

    gm = jnp.max(gl, axis=0, keepdims=True)
    ge = jnp.exp(gl - gm)
    gp = ge / jnp.sum(ge, axis=0, keepdims=True)
    p0 = gp[0:1, :]
    p1 = gp[1:2, :]
    g1 = (p1 > p0).astype(jnp.int32)
    g_prob = jnp.where(g1 > 0, p1, p0)

    row = jax.lax.broadcasted_iota(jnp.int32, el.shape, 0)
    in_group = (row // EPG) == g1
    neg = jnp.float32(-1e30)
    el_m = jnp.where(in_group, el, neg)
    em = jnp.max(el_m, axis=0, keepdims=True)
    ee = jnp.exp(el_m - em)
    ep = ee / jnp.sum(ee, axis=0, keepdims=True)
    p_exp = ep * g_prob

    rank = jnp.zeros_like(p_exp)
    for j in range(NUM_EXPERTS):
        pj = p_exp[j:j + 1, :]
        rank = rank + (pj > p_exp).astype(jnp.float32)
        rank = rank + ((pj == p_exp) & (row > j)).astype(jnp.float32)
    c_t = jnp.where(rank < TOP_K, p_exp, 0.0)
    c = c_t.T.astype(jnp.bfloat16)

    xb = xt.astype(jnp.bfloat16)
    big1 = jnp.dot(xb, w_in_ref[...], preferred_element_type=jnp.float32)
    hs = big1[:, :2 * HIDDEN]
    h1 = big1[:, 2 * HIDDEN:]
    a = hs[:, :HIDDEN]
    b = hs[:, HIDDEN:]
    hid = ((a * jax.nn.sigmoid(a)) * b).astype(jnp.bfloat16)
    a1 = h1[:, :HIDDEN]
    b1 = h1[:, HIDDEN:]
    h_all = ((a1 * jax.nn.sigmoid(a1)) * b1).astype(jnp.bfloat16)

    parts = [hid] + [h_all * c[:, e:e + 1] for e in range(NUM_EXPERTS)]
    scaled = jnp.concatenate(parts, axis=1)
    out_ref[...] = jnp.dot(scaled, w_out_ref[...],
                           preferred_element_type=jnp.float32)


def kernel(x, shared_in_w, shared_out_w, w1_shared_w, w2_expert_w,
           group_gate_w, expert_gate_w, group_bias, expert_bias):
    Bb, Tt, C = x.shape
    S = Bb * Tt
    flat = x.reshape(S, C)
    gates = jnp.concatenate([group_gate_w, expert_gate_w], axis=1)
    bias = jnp.concatenate([group_bias, expert_bias]).reshape(
        NUM_GROUPS + NUM_EXPERTS, 1)

    w_in = jnp.concatenate([shared_in_w, w1_shared_w],
                           axis=1).astype(jnp.bfloat16)
    w_out = jnp.concatenate(
        [shared_out_w, w2_expert_w.reshape(NUM_EXPERTS * HIDDEN, C)],
        axis=0).astype(jnp.bfloat16)

    grid = (S // TOKEN_TILE,)
    full = lambda *shape: pl.BlockSpec(shape, lambda i: (0,) * len(shape))

    out = pl.pallas_call(
        _moe_kernel,
        grid=grid,
        in_specs=[
            pl.BlockSpec((TOKEN_TILE, C), lambda i: (i, 0)),
            full(C, 4 * HIDDEN),
            full((1 + NUM_EXPERTS) * HIDDEN, C),
            full(C, NUM_GROUPS + NUM_EXPERTS),
            full(NUM_GROUPS + NUM_EXPERTS, 1),
        ],
        out_specs=pl.BlockSpec((TOKEN_TILE, C), lambda i: (i, 0)),
        out_shape=jax.ShapeDtypeStruct((S, C), jnp.float32),
    )(flat, w_in, w_out, gates, bias)

    return out.reshape(Bb, Tt, C)


# speedup vs baseline: 22.0161x; 1.0559x over previous
import jax
import jax.numpy as jnp
from jax.experimental import pallas as pl
from jax.experimental.pallas import tpu as pltpu

MODEL_DIM = 768
HIDDEN = 256
NUM_GROUPS = 2
EPG = 4
NUM_EXPERTS = NUM_GROUPS * EPG
TOP_K = 2
NG = NUM_GROUPS + NUM_EXPERTS

TOKEN_TILE = 1024


def _moe_kernel(x_ref, si_ref, so_ref, w1_ref, w2_ref, gg_ref, eg_ref,
                gb_ref, eb_ref, out_ref, w_in_s, w_out_s, gates_s, bias_s):
    i = pl.program_id(0)

    @pl.when(i == 0)
    def _prep():
        w_in_s[:, :2 * HIDDEN] = si_ref[...].astype(jnp.bfloat16)
        w_in_s[:, 2 * HIDDEN:] = w1_ref[...].astype(jnp.bfloat16)
        w_out_s[:HIDDEN, :] = so_ref[...].astype(jnp.bfloat16)
        w_out_s[HIDDEN:, :] = w2_ref[...].astype(jnp.bfloat16)
        gates_s[:, :NUM_GROUPS] = gg_ref[...]
        gates_s[:, NUM_GROUPS:] = eg_ref[...]
        bias_s[:NUM_GROUPS, :] = gb_ref[...]
        bias_s[NUM_GROUPS:, :] = eb_ref[...]

    xt = x_ref[...]

    logits_t = jax.lax.dot_general(
        gates_s[...], xt, (((0,), (1,)), ((), ())),
        preferred_element_type=jnp.float32)
    logits_t = logits_t + bias_s[...]
    gl = logits_t[:NUM_GROUPS, :]
    el = logits_t[NUM_GROUPS:, :]

    gm = jnp.max(gl, axis=0, keepdims=True)
    ge = jnp.exp(gl - gm)
    gp = ge / jnp.sum(ge, axis=0, keepdims=True)
    p0 = gp[0:1, :]
    p1 = gp[1:2, :]
    g1 = (p1 > p0).astype(jnp.int32)
    g_prob = jnp.where(g1 > 0, p1, p0)

    row = jax.lax.broadcasted_iota(jnp.int32, el.shape, 0)
    in_group = (row // EPG) == g1
    neg = jnp.float32(-1e30)
    el_m = jnp.where(in_group, el, neg)
    em = jnp.max(el_m, axis=0, keepdims=True)
    ee = jnp.exp(el_m - em)
    ep = ee / jnp.sum(ee, axis=0, keepdims=True)
    p_exp = ep * g_prob

    rank = jnp.zeros_like(p_exp)
    for j in range(NUM_EXPERTS):
        pj = p_exp[j:j + 1, :]
        rank = rank + (pj > p_exp).astype(jnp.float32)
        rank = rank + ((pj == p_exp) & (row > j)).astype(jnp.float32)
    c_t = jnp.where(rank < TOP_K, p_exp, 0.0)
    c = c_t.T.astype(jnp.bfloat16)

    xb = xt.astype(jnp.bfloat16)
    big1 = jnp.dot(xb, w_in_s[...], preferred_element_type=jnp.float32)
    hs = big1[:, :2 * HIDDEN]
    h1 = big1[:, 2 * HIDDEN:]
    a = hs[:, :HIDDEN]
    b = hs[:, HIDDEN:]
    hid = ((a * jax.nn.sigmoid(a)) * b).astype(jnp.bfloat16)
    a1 = h1[:, :HIDDEN]
    b1 = h1[:, HIDDEN:]
    h_all = ((a1 * jax.nn.sigmoid(a1)) * b1).astype(jnp.bfloat16)

    parts = [hid] + [h_all * c[:, e:e + 1] for e in range(NUM_EXPERTS)]
    scaled = jnp.concatenate(parts, axis=1)
    out_ref[...] = jnp.dot(scaled, w_out_s[...],
                           preferred_element_type=jnp.float32)


def kernel(x, shared_in_w, shared_out_w, w1_shared_w, w2_expert_w,
           group_gate_w, expert_gate_w, group_bias, expert_bias):
    Bb, Tt, C = x.shape
    S = Bb * Tt
    flat = x.reshape(S, C)
    w2_flat = w2_expert_w.reshape(NUM_EXPERTS * HIDDEN, C)
    gb = group_bias.reshape(NUM_GROUPS, 1)
    eb = expert_bias.reshape(NUM_EXPERTS, 1)

    grid = (S // TOKEN_TILE,)
    full = lambda *shape: pl.BlockSpec(shape, lambda i: (0,) * len(shape))

    out = pl.pallas_call(
        _moe_kernel,
        grid=grid,
        in_specs=[
            pl.BlockSpec((TOKEN_TILE, C), lambda i: (i, 0)),
            full(C, 2 * HIDDEN),
            full(HIDDEN, C),
            full(C, 2 * HIDDEN),
            full(NUM_EXPERTS * HIDDEN, C),
            full(C, NUM_GROUPS),
            full(C, NUM_EXPERTS),
            full(NUM_GROUPS, 1),
            full(NUM_EXPERTS, 1),
        ],
        out_specs=pl.BlockSpec((TOKEN_TILE, C), lambda i: (i, 0)),
        out_shape=jax.ShapeDtypeStruct((S, C), jnp.float32),
        scratch_shapes=[
            pltpu.VMEM((C, 4 * HIDDEN), jnp.bfloat16),
            pltpu.VMEM(((1 + NUM_EXPERTS) * HIDDEN, C), jnp.bfloat16),
            pltpu.VMEM((C, NG), jnp.float32),
            pltpu.VMEM((NG, 1), jnp.float32),
        ],
    )(flat, shared_in_w, shared_out_w, w1_shared_w, w2_flat,
      group_gate_w, expert_gate_w, gb, eb)

    return out.reshape(Bb, Tt, C)
